# baseline (speedup 1.0000x reference)
import functools

import jax
import jax.numpy as jnp
from jax import lax
from jax.experimental import pallas as pl
from jax.experimental.pallas import tpu as pltpu
from jax.experimental.pallas import tpu_sc as plsc

H = 8
NC = 2
NS = 16
NW = NC * NS
B = 400
W = 56
FB = 16


def _proj_body(x_ref, w_ref, q_ref, kv_ref, *, fqk, scaling):
    proj = lax.dot_general(x_ref[...], w_ref[...],
                           (((1,), (1,)), ((), ())),
                           preferred_element_type=jnp.float32)
    q_ref[...] = proj[:, :fqk] * scaling
    kv_ref[...] = proj[:, fqk:]


def _project(x, w, fqk, scaling):
    n, fin = x.shape
    tot = w.shape[0]
    bn = 1000 if n % 1000 == 0 else n
    f32 = jnp.float32
    return pl.pallas_call(
        functools.partial(_proj_body, fqk=fqk, scaling=scaling),
        grid=(n // bn,),
        in_specs=[
            pl.BlockSpec((bn, fin), lambda i: (i, 0)),
            pl.BlockSpec((tot, fin), lambda i: (0, 0)),
        ],
        out_specs=[
            pl.BlockSpec((bn, fqk), lambda i: (i, 0)),
            pl.BlockSpec((bn, tot - fqk), lambda i: (i, 0)),
        ],
        out_shape=[
            jax.ShapeDtypeStruct((n, fqk), f32),
            jax.ShapeDtypeStruct((n, tot - fqk), f32),
        ],
    )(x, w)


def _sc_body(q_hbm, kv_hbm, src_hbm, dest_hbm, zinit_hbm, out_hbm,
             src_blk, dest_blk, qbuf, kvbuf, flush_buf, flush_idx, ex_buf,
             tmp_ex, acc_sh, *, n, ew, nblk, roww, npad, dummy):
    cid = lax.axis_index("c")
    sid = lax.axis_index("s")
    wid = sid * NC + cid
    sl = npad // NS

    pltpu.sync_copy(zinit_hbm, acc_sh.at[pl.ds(sid * sl, sl)])
    plsc.subcore_barrier()

    iota = lax.iota(jnp.int32, 16)
    lanes_lt8 = iota < 8
    dummy_vec = jnp.full((16,), dummy, jnp.int32)
    zero16 = jnp.zeros((16,), jnp.float32)
    rot8 = jnp.bitwise_and(iota + 8, 15)

    flush_idx[...] = dummy_vec

    def flush(cur, count, accs):
        for j in range(8):
            flush_buf[count, pl.ds(j * 16, 16)] = accs[j]
        tmp_ex[...] = accs[8]
        folded = accs[8] + plsc.load_gather(tmp_ex, [rot8])
        flush_buf[count, pl.ds(8 * 16, 16)] = jnp.where(lanes_lt8, folded, 0.0)
        idxv = flush_idx[...]
        flush_idx[...] = jnp.where(iota == count,
                                   jnp.where(cur < 0, dummy, cur), idxv)
        count = count + 1

        @pl.when(count == FB)
        def _():
            pltpu.sync_copy(flush_buf, acc_sh.at[flush_idx], add=True)
            flush_idx[...] = dummy_vec

        return jnp.where(count == FB, 0, count)

    def group_compute(src_grp, rq, rkv, carry):
        cur, count, accs = carry
        neq = src_grp != cur
        nbnd = plsc.all_reduce_population_count(neq)[0]

        def fast(c):
            cur_, count_, ac = c
            ac = list(ac)
            rq0 = rq[0]
            rkvs = [rkv[j] for j in range(16)]
            qsl = [qbuf[rq0, pl.ds(h * 16, 16)] for h in range(H)]
            for p in range(8):
                aw = zero16
                for h in range(H):
                    ka_ = kvbuf[rkvs[2 * p], pl.ds(h * 16, 16)]
                    kb_ = kvbuf[rkvs[2 * p + 1], pl.ds(h * 16, 16)]
                    aw = jnp.where(iota == h, jnp.sum(qsl[h] * ka_), aw)
                    aw = jnp.where(iota == 8 + h, jnp.sum(qsl[h] * kb_), aw)
                ex2 = jnp.exp(aw) + 1e-8
                for h in range(H):
                    va_ = kvbuf[rkvs[2 * p], pl.ds(128 + h * 16, 16)]
                    vb_ = kvbuf[rkvs[2 * p + 1], pl.ds(128 + h * 16, 16)]
                    ac[h] = ac[h] + ex2[h] * va_ + ex2[8 + h] * vb_
                ac[8] = ac[8] + ex2
            return (cur_, count_, tuple(ac))

        def slow(c):
            rqs = [rq[j] for j in range(16)]
            rkvs = [rkv[j] for j in range(16)]
            for p in range(8):
                aw = zero16
                for h in range(H):
                    qa_ = qbuf[rqs[2 * p], pl.ds(h * 16, 16)]
                    ka_ = kvbuf[rkvs[2 * p], pl.ds(h * 16, 16)]
                    aw = jnp.where(iota == h, jnp.sum(qa_ * ka_), aw)
                    qb_ = qbuf[rqs[2 * p + 1], pl.ds(h * 16, 16)]
                    kb_ = kvbuf[rkvs[2 * p + 1], pl.ds(h * 16, 16)]
                    aw = jnp.where(iota == 8 + h, jnp.sum(qb_ * kb_), aw)
                ex_buf[pl.ds(p * 16, 16)] = jnp.exp(aw) + 1e-8
            cin = c
            for jj in range(16):
                cur_, count_, ac = cin
                is_new = src_grp[jj] != cur_
                count_ = lax.cond(
                    is_new,
                    lambda cc, cr=cur_, aa=ac: flush(cr, cc, aa),
                    lambda cc: cc,
                    count_)
                keep = jnp.where(is_new, 0.0, 1.0)
                ac = tuple(a * keep for a in ac)
                ex2 = ex_buf[pl.ds((jj // 2) * 16, 16)]
                lane = (jj % 2) * 8
                new_ac = []
                for h in range(H):
                    vs = kvbuf[rkvs[jj], pl.ds(128 + h * 16, 16)]
                    new_ac.append(ac[h] + ex2[lane + h] * vs)
                if jj % 2 == 0:
                    exm = jnp.where(lanes_lt8, ex2, 0.0)
                else:
                    exm = jnp.where(lanes_lt8, 0.0, ex2)
                new_ac.append(ac[8] + exm)
                cin = (src_grp[jj], count_, tuple(new_ac))
            return cin

        return lax.cond(nbnd == 0, fast, slow, (cur, count, accs))

    base = wid * ew

    def block_body(blk, carry):
        off = pl.multiple_of(base + blk * B, 8)
        pltpu.sync_copy(src_hbm.at[pl.ds(off, B)], src_blk)
        pltpu.sync_copy(dest_hbm.at[pl.ds(off, B)], dest_blk)
        s_lo = src_blk[pl.ds(0, 16)][0]
        s_hi = src_blk[pl.ds(B - 16, 16)][15]
        d_lo = dest_blk[pl.ds(0, 16)][0]
        d_hi = dest_blk[pl.ds(B - 16, 16)][15]
        base_q = jnp.minimum(s_lo, n - W)
        base_kv = jnp.minimum(d_lo, n - W)
        ok = jnp.logical_and(s_hi - base_q < W, d_hi - base_kv < W)

        def range_path(c):
            pltpu.sync_copy(q_hbm.at[pl.ds(base_q, W)], qbuf)
            pltpu.sync_copy(kv_hbm.at[pl.ds(base_kv, W)], kvbuf)

            def gbody(g, cc):
                goff = pl.multiple_of(g * 16, 16)
                src_grp = src_blk[pl.ds(goff, 16)]
                dest_grp = dest_blk[pl.ds(goff, 16)]
                return group_compute(src_grp, src_grp - base_q,
                                     dest_grp - base_kv, cc)

            return lax.fori_loop(0, B // 16, gbody, c)

        def gather_path(c):
            def gbody(g, cc):
                goff = pl.multiple_of(g * 16, 16)
                sidx = src_blk.at[pl.ds(goff, 16)]
                didx = dest_blk.at[pl.ds(goff, 16)]
                pltpu.sync_copy(q_hbm.at[sidx], qbuf.at[pl.ds(0, 16)])
                pltpu.sync_copy(kv_hbm.at[didx], kvbuf.at[pl.ds(0, 16)])
                src_grp = src_blk[pl.ds(goff, 16)]
                return group_compute(src_grp, iota, iota, cc)

            return lax.fori_loop(0, B // 16, gbody, c)

        return lax.cond(ok, range_path, gather_path, carry)

    carry0 = (jnp.int32(-1), jnp.int32(0), tuple(zero16 for _ in range(9)))
    cur, count, accs = lax.fori_loop(0, nblk, block_body, carry0)

    flush(cur, count, accs)
    pltpu.sync_copy(flush_buf, acc_sh.at[flush_idx], add=True)

    plsc.subcore_barrier()
    pltpu.sync_copy(acc_sh.at[pl.ds(sid * sl, sl)],
                    out_hbm.at[cid, pl.ds(sid * sl, sl)])


def _sc_attend(q, kv, src, dest):
    n = q.shape[0]
    e = src.shape[0]
    roww = 144
    ew = e // NW
    nblk = ew // B
    npad = NS * 8 * -(-(n + 1) // (NS * 8))
    sl = npad // NS
    f32 = jnp.float32
    zinit = jnp.zeros((sl, roww), f32)

    body = functools.partial(_sc_body, n=n, ew=ew, nblk=nblk, roww=roww,
                             npad=npad, dummy=n)
    mesh = plsc.VectorSubcoreMesh(core_axis_name="c", subcore_axis_name="s",
                                  num_cores=NC, num_subcores=NS)
    return pl.kernel(
        body,
        out_type=jax.ShapeDtypeStruct((NC, npad, roww), f32),
        mesh=mesh,
        compiler_params=pltpu.CompilerParams(
            needs_layout_passes=False, use_tc_tiling_on_sc=False),
        scratch_types=[
            pltpu.VMEM((B,), jnp.int32),
            pltpu.VMEM((B,), jnp.int32),
            pltpu.VMEM((W, 128), f32),
            pltpu.VMEM((W, 256), f32),
            pltpu.VMEM((FB, roww), f32),
            pltpu.VMEM((FB,), jnp.int32),
            pltpu.VMEM((128,), f32),
            pltpu.VMEM((16,), f32),
            pltpu.VMEM_SHARED((npad, roww), f32),
        ],
    )(q, kv, src, dest, zinit)


def _norm_body(p_ref, out_ref):
    t = p_ref[0] + p_ref[1]
    for h in range(H):
        a = t[:, h * 16:(h + 1) * 16]
        s = t[:, 128 + h:129 + h]
        out_ref[:, h * 16:(h + 1) * 16] = jnp.where(s > 0, a / s, 0.0)


def _normalize(partial, n, fv):
    npad, roww = partial.shape[1], partial.shape[2]
    bn = 1000 if n % 1000 == 0 else n
    return pl.pallas_call(
        _norm_body,
        grid=(n // bn,),
        in_specs=[pl.BlockSpec((NC, bn, roww), lambda i: (0, i, 0))],
        out_specs=pl.BlockSpec((bn, fv), lambda i: (i, 0)),
        out_shape=jax.ShapeDtypeStruct((n, fv), jnp.float32),
    )(partial)


def kernel(x, batch, ei, W_):
    n = x.shape[0]
    tot = W_.shape[0]
    fqk = tot // 3
    fv = tot - 2 * fqk
    scaling = float(fqk // H) ** (-0.5)
    src = ei[0]
    dest = ei[1]
    q, kv = _project(x, W_, fqk, scaling)
    partial = _sc_attend(q, kv, src, dest)
    return _normalize(partial, n, fv)


# speedup vs baseline: 1.5598x; 1.5598x over previous
import functools

import jax
import jax.numpy as jnp
from jax import lax
from jax.experimental import pallas as pl
from jax.experimental.pallas import tpu as pltpu
from jax.experimental.pallas import tpu_sc as plsc

H = 8
NC = 2
NS = 16
NW = NC * NS
B = 400
W = 56
FB = 16


def _proj_body(x_ref, w_ref, q_ref, kv_ref, *, fqk, scaling):
    proj = lax.dot_general(x_ref[...], w_ref[...],
                           (((1,), (1,)), ((), ())),
                           preferred_element_type=jnp.float32)
    q_ref[...] = proj[:, :fqk] * scaling
    kv_ref[...] = proj[:, fqk:]


def _project(x, w, fqk, scaling):
    n, fin = x.shape
    tot = w.shape[0]
    bn = 1000 if n % 1000 == 0 else n
    f32 = jnp.float32
    return pl.pallas_call(
        functools.partial(_proj_body, fqk=fqk, scaling=scaling),
        grid=(n // bn,),
        in_specs=[
            pl.BlockSpec((bn, fin), lambda i: (i, 0)),
            pl.BlockSpec((tot, fin), lambda i: (0, 0)),
        ],
        out_specs=[
            pl.BlockSpec((bn, fqk), lambda i: (i, 0)),
            pl.BlockSpec((bn, tot - fqk), lambda i: (i, 0)),
        ],
        out_shape=[
            jax.ShapeDtypeStruct((n, fqk), f32),
            jax.ShapeDtypeStruct((n, tot - fqk), f32),
        ],
    )(x, w)


def _sc_body(q_hbm, kv_hbm, src_hbm, dest_hbm, zinit_hbm, out_hbm,
             src_blk, dest_blk, qbuf, kvbuf, flush_buf, flush_idx, ex_buf,
             tmp_ex, acc_sh, *, n, ew, nblk, roww, npad, dummy):
    cid = lax.axis_index("c")
    sid = lax.axis_index("s")
    wid = sid * NC + cid
    sl = npad // NS

    pltpu.sync_copy(zinit_hbm, acc_sh.at[pl.ds(sid * sl, sl)])
    plsc.subcore_barrier()

    iota = lax.iota(jnp.int32, 16)
    lanes_lt8 = iota < 8
    dummy_vec = jnp.full((16,), dummy, jnp.int32)
    zero16 = jnp.zeros((16,), jnp.float32)
    rot8 = jnp.bitwise_and(iota + 8, 15)

    flush_idx[...] = dummy_vec

    def flush(cur, count, accs):
        for j in range(8):
            flush_buf[count, pl.ds(j * 16, 16)] = accs[j]
        tmp_ex[...] = accs[8]
        folded = accs[8] + plsc.load_gather(tmp_ex, [rot8])
        flush_buf[count, pl.ds(8 * 16, 16)] = jnp.where(lanes_lt8, folded, 0.0)
        idxv = flush_idx[...]
        flush_idx[...] = jnp.where(iota == count,
                                   jnp.where(cur < 0, dummy, cur), idxv)
        count = count + 1

        @pl.when(count == FB)
        def _():
            pltpu.sync_copy(flush_buf, acc_sh.at[flush_idx], add=True)
            flush_idx[...] = dummy_vec

        return jnp.where(count == FB, 0, count)

    def group_compute(src_grp, rq, rkv, carry):
        rqs = [rq[j] for j in range(16)]
        rkvs = [rkv[j] for j in range(16)]

        for p in range(8):
            aw = zero16
            for h in range(H):
                qa_ = qbuf[rqs[2 * p], pl.ds(h * 16, 16)]
                ka_ = kvbuf[rkvs[2 * p], pl.ds(h * 16, 16)]
                aw = jnp.where(iota == h, jnp.sum(qa_ * ka_), aw)
                qb_ = qbuf[rqs[2 * p + 1], pl.ds(h * 16, 16)]
                kb_ = kvbuf[rkvs[2 * p + 1], pl.ds(h * 16, 16)]
                aw = jnp.where(iota == 8 + h, jnp.sum(qb_ * kb_), aw)
            ex_buf[pl.ds(p * 16, 16)] = jnp.exp(aw) + 1e-8

        cur, count, accs = carry
        neq = src_grp != cur
        nbnd = plsc.all_reduce_population_count(neq)[0]

        def fast(c):
            cur_, count_, ac = c
            ac = list(ac)
            for p in range(8):
                ex2 = ex_buf[pl.ds(p * 16, 16)]
                for h in range(H):
                    va_ = kvbuf[rkvs[2 * p], pl.ds(128 + h * 16, 16)]
                    vb_ = kvbuf[rkvs[2 * p + 1], pl.ds(128 + h * 16, 16)]
                    ac[h] = ac[h] + ex2[h] * va_ + ex2[8 + h] * vb_
                ac[8] = ac[8] + ex2
            return (cur_, count_, tuple(ac))

        def slow(c):
            cin = c
            for jj in range(16):
                cur_, count_, ac = cin
                is_new = src_grp[jj] != cur_
                count_ = lax.cond(
                    is_new,
                    lambda cc, cr=cur_, aa=ac: flush(cr, cc, aa),
                    lambda cc: cc,
                    count_)
                keep = jnp.where(is_new, 0.0, 1.0)
                ac = tuple(a * keep for a in ac)
                ex2 = ex_buf[pl.ds((jj // 2) * 16, 16)]
                lane = (jj % 2) * 8
                new_ac = []
                for h in range(H):
                    vs = kvbuf[rkvs[jj], pl.ds(128 + h * 16, 16)]
                    new_ac.append(ac[h] + ex2[lane + h] * vs)
                if jj % 2 == 0:
                    exm = jnp.where(lanes_lt8, ex2, 0.0)
                else:
                    exm = jnp.where(lanes_lt8, 0.0, ex2)
                new_ac.append(ac[8] + exm)
                cin = (src_grp[jj], count_, tuple(new_ac))
            return cin

        return lax.cond(nbnd == 0, fast, slow, (cur, count, accs))

    base = wid * ew

    def block_body(blk, carry):
        off = pl.multiple_of(base + blk * B, 8)
        pltpu.sync_copy(src_hbm.at[pl.ds(off, B)], src_blk)
        pltpu.sync_copy(dest_hbm.at[pl.ds(off, B)], dest_blk)
        s_lo = src_blk[pl.ds(0, 16)][0]
        s_hi = src_blk[pl.ds(B - 16, 16)][15]
        d_lo = dest_blk[pl.ds(0, 16)][0]
        d_hi = dest_blk[pl.ds(B - 16, 16)][15]
        base_q = jnp.minimum(s_lo, n - W)
        base_kv = jnp.minimum(d_lo, n - W)
        ok = jnp.logical_and(s_hi - base_q < W, d_hi - base_kv < W)

        def range_path(c):
            pltpu.sync_copy(q_hbm.at[pl.ds(base_q, W)], qbuf)
            pltpu.sync_copy(kv_hbm.at[pl.ds(base_kv, W)], kvbuf)

            def gbody(g, cc):
                goff = pl.multiple_of(g * 16, 16)
                src_grp = src_blk[pl.ds(goff, 16)]
                dest_grp = dest_blk[pl.ds(goff, 16)]
                return group_compute(src_grp, src_grp - base_q,
                                     dest_grp - base_kv, cc)

            return lax.fori_loop(0, B // 16, gbody, c)

        def gather_path(c):
            def gbody(g, cc):
                goff = pl.multiple_of(g * 16, 16)
                sidx = src_blk.at[pl.ds(goff, 16)]
                didx = dest_blk.at[pl.ds(goff, 16)]
                pltpu.sync_copy(q_hbm.at[sidx], qbuf.at[pl.ds(0, 16)])
                pltpu.sync_copy(kv_hbm.at[didx], kvbuf.at[pl.ds(0, 16)])
                src_grp = src_blk[pl.ds(goff, 16)]
                return group_compute(src_grp, iota, iota, cc)

            return lax.fori_loop(0, B // 16, gbody, c)

        return lax.cond(ok, range_path, gather_path, carry)

    carry0 = (jnp.int32(-1), jnp.int32(0), tuple(zero16 for _ in range(9)))
    cur, count, accs = lax.fori_loop(0, nblk, block_body, carry0)

    flush(cur, count, accs)
    pltpu.sync_copy(flush_buf, acc_sh.at[flush_idx], add=True)

    plsc.subcore_barrier()
    pltpu.sync_copy(acc_sh.at[pl.ds(sid * sl, sl)],
                    out_hbm.at[cid, pl.ds(sid * sl, sl)])


def _sc_attend(q, kv, src, dest):
    n = q.shape[0]
    e = src.shape[0]
    roww = 144
    ew = e // NW
    nblk = ew // B
    npad = NS * 8 * -(-(n + 1) // (NS * 8))
    sl = npad // NS
    f32 = jnp.float32
    zinit = jnp.zeros((sl, roww), f32)

    body = functools.partial(_sc_body, n=n, ew=ew, nblk=nblk, roww=roww,
                             npad=npad, dummy=n)
    mesh = plsc.VectorSubcoreMesh(core_axis_name="c", subcore_axis_name="s",
                                  num_cores=NC, num_subcores=NS)
    return pl.kernel(
        body,
        out_type=jax.ShapeDtypeStruct((NC, npad, roww), f32),
        mesh=mesh,
        compiler_params=pltpu.CompilerParams(
            needs_layout_passes=False, use_tc_tiling_on_sc=False),
        scratch_types=[
            pltpu.VMEM((B,), jnp.int32),
            pltpu.VMEM((B,), jnp.int32),
            pltpu.VMEM((W, 128), f32),
            pltpu.VMEM((W, 256), f32),
            pltpu.VMEM((FB, roww), f32),
            pltpu.VMEM((FB,), jnp.int32),
            pltpu.VMEM((128,), f32),
            pltpu.VMEM((16,), f32),
            pltpu.VMEM_SHARED((npad, roww), f32),
        ],
    )(q, kv, src, dest, zinit)


def _norm_body(p_ref, out_ref):
    t = p_ref[0] + p_ref[1]
    for h in range(H):
        a = t[:, h * 16:(h + 1) * 16]
        s = t[:, 128 + h:129 + h]
        out_ref[:, h * 16:(h + 1) * 16] = jnp.where(s > 0, a / s, 0.0)


def _normalize(partial, n, fv):
    npad, roww = partial.shape[1], partial.shape[2]
    bn = 1000 if n % 1000 == 0 else n
    return pl.pallas_call(
        _norm_body,
        grid=(n // bn,),
        in_specs=[pl.BlockSpec((NC, bn, roww), lambda i: (0, i, 0))],
        out_specs=pl.BlockSpec((bn, fv), lambda i: (i, 0)),
        out_shape=jax.ShapeDtypeStruct((n, fv), jnp.float32),
    )(partial)


def kernel(x, batch, ei, W_):
    n = x.shape[0]
    tot = W_.shape[0]
    fqk = tot // 3
    fv = tot - 2 * fqk
    scaling = float(fqk // H) ** (-0.5)
    src = ei[0]
    dest = ei[1]
    q, kv = _project(x, W_, fqk, scaling)
    partial = _sc_attend(q, kv, src, dest)
    return _normalize(partial, n, fv)
